# baseline (device time: 126371 ns/iter reference)
import jax
import jax.numpy as jnp
from jax import lax
from jax.experimental import pallas as pl
from jax.experimental.pallas import tpu as pltpu

N_DEV = 16
HEADS_PER = 4
DH = 64
SQ_L = 128
SKV = 128
B = 2
E = 512
HL = HEADS_PER * DH


def kernel(x, Wq, K_ext, V_ext, Wo):
    xb = x.astype(jnp.bfloat16)
    wq = Wq.astype(jnp.bfloat16)
    wot = Wo.T.astype(jnp.bfloat16)
    kt = jnp.transpose(K_ext, (2, 0, 1, 3)).astype(jnp.bfloat16)
    vt = jnp.transpose(V_ext, (2, 0, 1, 3)).astype(jnp.bfloat16)

    def body(x_ref, wq_ref, wot_ref, kt_ref, vt_ref, out_ref,
             buf, acc, qs, cs, send_sems, recv_sems):
        my = lax.axis_index("i")
        left = lax.rem(my - 1 + N_DEV, N_DEV)
        right = lax.rem(my + 1, N_DEV)

        bsem = pltpu.get_barrier_semaphore()
        for nbr in (left, right):
            pl.semaphore_signal(bsem, inc=1, device_id=(nbr,),
                                device_id_type=pl.DeviceIdType.MESH)
        pl.semaphore_wait(bsem, 2)

        buf[my, 0] = wq_ref[...]
        buf[my, 1] = wot_ref[...]
        acc[...] = jnp.zeros_like(acc)

        x2d = x_ref[...].reshape(B * SQ_L, E)

        q_ids = my * SQ_L + lax.broadcasted_iota(jnp.int32, (SQ_L, SKV), 0)
        k_ids = lax.broadcasted_iota(jnp.int32, (SQ_L, SKV), 1)
        qb = q_ids // 64
        kb = k_ids // 64
        mask = (qb == kb) | (kb == 0) | (lax.rem(qb + kb, 3) == 0)
        neg = jnp.where(mask, 0.0, -1e9).astype(jnp.float32)

        def compute_block(j):
            wq_j = buf[j, 0]
            q = lax.dot_general(x2d, wq_j, (((1,), (0,)), ((), ())),
                                preferred_element_type=jnp.float32)
            qs[...] = q.astype(jnp.bfloat16)
            for b in range(B):
                for hh in range(HEADS_PER):
                    h_idx = j * HEADS_PER + hh
                    qbh = qs[b * SQ_L:(b + 1) * SQ_L, hh * DH:(hh + 1) * DH]
                    kbh = kt_ref[h_idx, b]
                    s = lax.dot_general(qbh, kbh, (((1,), (1,)), ((), ())),
                                        preferred_element_type=jnp.float32)
                    s = s * 0.125 + neg
                    m = jnp.max(s, axis=1, keepdims=True)
                    e = jnp.exp(s - m)
                    w = (e / jnp.sum(e, axis=1, keepdims=True)).astype(jnp.bfloat16)
                    vbh = vt_ref[h_idx, b]
                    c = lax.dot_general(w, vbh, (((1,), (0,)), ((), ())),
                                        preferred_element_type=jnp.float32)
                    cs[b * SQ_L:(b + 1) * SQ_L, hh * DH:(hh + 1) * DH] = (
                        c.astype(jnp.bfloat16))
            acc[...] = acc[...] + lax.dot_general(
                cs[...], buf[j, 1], (((1,), (1,)), ((), ())),
                preferred_element_type=jnp.float32)

        for h in range(N_DEV - 1):
            slot = lax.rem(my - h + N_DEV, N_DEV)
            rdma = pltpu.make_async_remote_copy(
                src_ref=buf.at[slot],
                dst_ref=buf.at[slot],
                send_sem=send_sems.at[h],
                recv_sem=recv_sems.at[h],
                device_id=(right,),
                device_id_type=pl.DeviceIdType.MESH,
            )
            rdma.start()
            compute_block(slot)
            rdma.wait_send()
            rdma.wait_recv()
        compute_block(lax.rem(my + 1, N_DEV))

        out_ref[...] = acc[...].reshape(B, SQ_L, E)

    return pl.pallas_call(
        body,
        out_shape=jax.ShapeDtypeStruct((B, SQ_L, E), jnp.float32),
        in_specs=[pl.BlockSpec(memory_space=pltpu.VMEM)] * 5,
        out_specs=pl.BlockSpec(memory_space=pltpu.VMEM),
        scratch_shapes=[
            pltpu.VMEM((N_DEV, 2, E, HL), jnp.bfloat16),
            pltpu.VMEM((B * SQ_L, E), jnp.float32),
            pltpu.VMEM((B * SQ_L, HL), jnp.bfloat16),
            pltpu.VMEM((B * SQ_L, HL), jnp.bfloat16),
            pltpu.SemaphoreType.DMA((N_DEV - 1,)),
            pltpu.SemaphoreType.DMA((N_DEV - 1,)),
        ],
        compiler_params=pltpu.CompilerParams(collective_id=0),
    )(xb, wq, wot, kt, vt)


# device time: 86568 ns/iter; 1.4598x vs baseline; 1.4598x over previous
import jax
import jax.numpy as jnp
from jax import lax
from jax.experimental import pallas as pl
from jax.experimental.pallas import tpu as pltpu

N_DEV = 16
HEADS_PER = 4
DH = 64
SQ_L = 128
SKV = 128
B = 2
E = 512
HL = HEADS_PER * DH


def kernel(x, Wq, K_ext, V_ext, Wo):
    xb = x.astype(jnp.bfloat16)
    wq = Wq.astype(jnp.bfloat16)
    wot = Wo.T.astype(jnp.bfloat16)
    kt = jnp.transpose(K_ext, (2, 0, 1, 3)).astype(jnp.bfloat16)
    vt = jnp.transpose(V_ext, (2, 0, 1, 3)).astype(jnp.bfloat16)

    def body(x_ref, wq_ref, wot_ref, kt_ref, vt_ref, out_ref,
             buf, acc, qs, cs, send_sems, recv_sems, lsend_sems, lrecv_sems):
        my = lax.axis_index("i")
        left = lax.rem(my - 1 + N_DEV, N_DEV)
        right = lax.rem(my + 1, N_DEV)

        bsem = pltpu.get_barrier_semaphore()
        for nbr in (left, right):
            pl.semaphore_signal(bsem, inc=1, device_id=(nbr,),
                                device_id_type=pl.DeviceIdType.MESH)
        pl.semaphore_wait(bsem, 2)

        buf[my, 0] = wq_ref[...]
        buf[my, 1] = wot_ref[...]
        acc[...] = jnp.zeros_like(acc)

        x2d = x_ref[...].reshape(B * SQ_L, E)

        q_ids = my * SQ_L + lax.broadcasted_iota(jnp.int32, (SQ_L, SKV), 0)
        k_ids = lax.broadcasted_iota(jnp.int32, (SQ_L, SKV), 1)
        qb = q_ids // 64
        kb = k_ids // 64
        mask = (qb == kb) | (kb == 0) | (lax.rem(qb + kb, 3) == 0)
        neg = jnp.where(mask, 0.0, -1e9).astype(jnp.float32)

        def compute_block(j):
            wq_j = buf[j, 0]
            q = lax.dot_general(x2d, wq_j, (((1,), (0,)), ((), ())),
                                preferred_element_type=jnp.float32)
            qs[...] = q.astype(jnp.bfloat16)
            for b in range(B):
                for hh in range(HEADS_PER):
                    h_idx = j * HEADS_PER + hh
                    qbh = qs[b * SQ_L:(b + 1) * SQ_L, hh * DH:(hh + 1) * DH]
                    kbh = kt_ref[h_idx, b]
                    s = lax.dot_general(qbh, kbh, (((1,), (1,)), ((), ())),
                                        preferred_element_type=jnp.float32)
                    s = s * 0.125 + neg
                    m = jnp.max(s, axis=1, keepdims=True)
                    e = jnp.exp(s - m)
                    w = (e / jnp.sum(e, axis=1, keepdims=True)).astype(jnp.bfloat16)
                    vbh = vt_ref[h_idx, b]
                    c = lax.dot_general(w, vbh, (((1,), (0,)), ((), ())),
                                        preferred_element_type=jnp.float32)
                    cs[b * SQ_L:(b + 1) * SQ_L, hh * DH:(hh + 1) * DH] = (
                        c.astype(jnp.bfloat16))
            acc[...] = acc[...] + lax.dot_general(
                cs[...], buf[j, 1], (((1,), (1,)), ((), ())),
                preferred_element_type=jnp.float32)

        R_HOPS = N_DEV // 2
        L_HOPS = N_DEV - 1 - R_HOPS

        def hop(direction_dev, slot, s_sems, r_sems, h):
            rdma = pltpu.make_async_remote_copy(
                src_ref=buf.at[slot],
                dst_ref=buf.at[slot],
                send_sem=s_sems.at[h],
                recv_sem=r_sems.at[h],
                device_id=(direction_dev,),
                device_id_type=pl.DeviceIdType.MESH,
            )
            rdma.start()
            return rdma

        for h in range(R_HOPS):
            r_rdma = hop(right, lax.rem(my - h + N_DEV, N_DEV),
                         send_sems, recv_sems, h)
            l_rdma = None
            if h < L_HOPS:
                l_rdma = hop(left, lax.rem(my + h, N_DEV),
                             lsend_sems, lrecv_sems, h)
            if h == 0:
                compute_block(my)
            else:
                compute_block(lax.rem(my - h + N_DEV, N_DEV))
                compute_block(lax.rem(my + min(h, L_HOPS), N_DEV))
            r_rdma.wait_send()
            r_rdma.wait_recv()
            if l_rdma is not None:
                l_rdma.wait_send()
                l_rdma.wait_recv()
        compute_block(lax.rem(my + R_HOPS, N_DEV))

        out_ref[...] = acc[...].reshape(B, SQ_L, E)

    return pl.pallas_call(
        body,
        out_shape=jax.ShapeDtypeStruct((B, SQ_L, E), jnp.float32),
        in_specs=[pl.BlockSpec(memory_space=pltpu.VMEM)] * 5,
        out_specs=pl.BlockSpec(memory_space=pltpu.VMEM),
        scratch_shapes=[
            pltpu.VMEM((N_DEV, 2, E, HL), jnp.bfloat16),
            pltpu.VMEM((B * SQ_L, E), jnp.float32),
            pltpu.VMEM((B * SQ_L, HL), jnp.bfloat16),
            pltpu.VMEM((B * SQ_L, HL), jnp.bfloat16),
            pltpu.SemaphoreType.DMA((N_DEV // 2,)),
            pltpu.SemaphoreType.DMA((N_DEV // 2,)),
            pltpu.SemaphoreType.DMA((N_DEV // 2 - 1,)),
            pltpu.SemaphoreType.DMA((N_DEV // 2 - 1,)),
        ],
        compiler_params=pltpu.CompilerParams(collective_id=0),
    )(xb, wq, wot, kt, vt)
